# baseline (speedup 1.0000x reference)
import functools

import jax
import jax.numpy as jnp
from jax import lax
from jax.experimental import pallas as pl
from jax.experimental.pallas import tpu as pltpu
from jax.experimental.pallas import tpu_sc as plsc

D_MODEL = 64
MAX_LEN = 200
BATCH = 4096
SEQ = 200
LANES = 128


@functools.lru_cache(maxsize=None)
def _make_sc_gather(V, B):
    NC, NS = 1, 16
    NW = NC * NS
    assert B % (8 * NW) == 0
    b_per_w = B // NW
    mesh = plsc.VectorSubcoreMesh(
        core_axis_name="c", subcore_axis_name="s", num_cores=NC, num_subcores=NS
    )

    @functools.partial(
        pl.kernel,
        mesh=mesh,
        out_type=jax.ShapeDtypeStruct((B, LANES), jnp.float32),
        scratch_types=[
            pltpu.VMEM((b_per_w,), jnp.int32),
            pltpu.VMEM((b_per_w, LANES), jnp.float32),
            pltpu.SemaphoreType.DMA,
        ],
    )
    def gather(table_hbm, idx_hbm, out_hbm, idx_v, rows_v, sem):
        wid = lax.axis_index("s") * NC + lax.axis_index("c")
        base = wid * b_per_w
        pltpu.sync_copy(idx_hbm.at[pl.ds(base, b_per_w)], idx_v)
        for g in range(b_per_w // 16):
            sl = pl.ds(g * 16, 16)
            idx_v[sl] = jnp.clip(idx_v[sl], 0, V - 1)
        pltpu.async_copy(table_hbm.at[idx_v], rows_v, sem).wait()
        pltpu.sync_copy(rows_v, out_hbm.at[pl.ds(base, b_per_w)])

    return gather


def _add_body(x_ref, pos_ref, o_ref, pos_t_ref):
    @pl.when(pl.program_id(0) == 0)
    def _():
        pos_t_ref[...] = pos_ref[:, :D_MODEL].T

    o_ref[...] = x_ref[...] + pos_t_ref[...][None, :, :]


def _tc_add(xt, pos, seq_per_block=5):
    S, D, B = xt.shape
    grid = (S // seq_per_block,)
    return pl.pallas_call(
        _add_body,
        grid=grid,
        in_specs=[
            pl.BlockSpec((seq_per_block, D, B), lambda i: (i, 0, 0)),
            pl.BlockSpec((B, LANES), lambda i: (0, 0)),
        ],
        out_specs=pl.BlockSpec((seq_per_block, D, B), lambda i: (i, 0, 0)),
        out_shape=jax.ShapeDtypeStruct((S, D, B), xt.dtype),
        scratch_shapes=[pltpu.VMEM((D, B), jnp.float32)],
    )(xt, pos)


def kernel(x, pe, step_indices):
    idx = step_indices.reshape(-1).astype(jnp.int32)
    table = jnp.pad(pe[0], ((0, 0), (0, LANES - D_MODEL)))
    pos = _make_sc_gather(MAX_LEN, BATCH)(table, idx)
    xt = x.transpose(1, 2, 0)
    out_t = _tc_add(xt, pos)
    return out_t.transpose(2, 0, 1)


# speedup vs baseline: 1.0123x; 1.0123x over previous
import functools

import jax
import jax.numpy as jnp
from jax import lax
from jax.experimental import pallas as pl
from jax.experimental.pallas import tpu as pltpu
from jax.experimental.pallas import tpu_sc as plsc

D_MODEL = 64
MAX_LEN = 200
BATCH = 4096
SEQ = 200
LANES = 128


@functools.lru_cache(maxsize=None)
def _make_sc_gather(V, B):
    NC, NS = 1, 16
    NW = NC * NS
    assert B % (8 * NW) == 0
    b_per_w = B // NW
    mesh = plsc.VectorSubcoreMesh(
        core_axis_name="c", subcore_axis_name="s", num_cores=NC, num_subcores=NS
    )

    @functools.partial(
        pl.kernel,
        mesh=mesh,
        out_type=jax.ShapeDtypeStruct((B, LANES), jnp.float32),
        scratch_types=[
            pltpu.VMEM((b_per_w,), jnp.int32),
            pltpu.VMEM((b_per_w, LANES), jnp.float32),
            pltpu.SemaphoreType.DMA,
        ],
    )
    def gather(table_hbm, idx_hbm, out_hbm, idx_v, rows_v, sem):
        wid = lax.axis_index("s") * NC + lax.axis_index("c")
        base = wid * b_per_w
        pltpu.sync_copy(idx_hbm.at[pl.ds(base, b_per_w)], idx_v)
        for g in range(b_per_w // 16):
            sl = pl.ds(g * 16, 16)
            idx_v[sl] = jnp.clip(idx_v[sl], 0, V - 1)
        pltpu.async_copy(table_hbm.at[idx_v], rows_v, sem).wait()
        pltpu.sync_copy(rows_v, out_hbm.at[pl.ds(base, b_per_w)])

    return gather


def _add_body(x_ref, pos_ref, o_ref, pos_t_ref):
    @pl.when(pl.program_id(0) == 0)
    def _():
        pos_t_ref[...] = pos_ref[:, :D_MODEL].T

    o_ref[...] = x_ref[...] + pos_t_ref[...][None, :, :]


def _tc_add(xt, pos, seq_per_block=10):
    S, D, B = xt.shape
    grid = (S // seq_per_block,)
    return pl.pallas_call(
        _add_body,
        grid=grid,
        in_specs=[
            pl.BlockSpec((seq_per_block, D, B), lambda i: (i, 0, 0)),
            pl.BlockSpec((B, LANES), lambda i: (0, 0)),
        ],
        out_specs=pl.BlockSpec((seq_per_block, D, B), lambda i: (i, 0, 0)),
        out_shape=jax.ShapeDtypeStruct((S, D, B), xt.dtype),
        scratch_shapes=[pltpu.VMEM((D, B), jnp.float32)],
    )(xt, pos)


def kernel(x, pe, step_indices):
    idx = step_indices.reshape(-1).astype(jnp.int32)
    table = jnp.pad(pe[0], ((0, 0), (0, LANES - D_MODEL)))
    pos = _make_sc_gather(MAX_LEN, BATCH)(table, idx)
    xt = x.transpose(1, 2, 0)
    out_t = _tc_add(xt, pos)
    return out_t.transpose(2, 0, 1)
